# baseline (speedup 1.0000x reference)
import jax
import jax.numpy as jnp
from jax.experimental import pallas as pl
from jax.experimental.pallas import tpu as pltpu
from jax.experimental.pallas import tpu_sc as plsc

_K = 8
_B = 16
_R1 = 1024
_R2 = 512
_C2 = 512
_R3 = 512


def _rows_normalized(x):
    n = jnp.sqrt(jnp.sum(x * x, axis=-1, keepdims=True))
    return x / jnp.maximum(n, 1e-12)


def _embed_body(ct_ref, cw1_ref, cb1_ref, cw2_ref, cb2_ref,
                pw1_ref, pb1_ref, pw2_ref, pb2_ref,
                mwce_ref, mwcol_ref, mwpos_ref, mb_ref,
                ew1a_ref, ew1b_ref, eb1_ref, eg1_ref, ebt1_ref,
                colors_ref, pos_ref, cls_ref,
                ea_ref, ebm_ref, u_ref, v_ref):
    f32 = jnp.float32
    cls = cls_ref[...]
    onehot = (cls == jax.lax.broadcasted_iota(
        jnp.int32, (1, ct_ref.shape[0]), 1)).astype(f32)
    ce = _rows_normalized(jnp.dot(onehot, ct_ref[...],
                                  preferred_element_type=f32))
    ch = jnp.maximum(jnp.dot(colors_ref[...], cw1_ref[...],
                             preferred_element_type=f32) + cb1_ref[...], 0.0)
    col = _rows_normalized(jnp.dot(ch, cw2_ref[...],
                                   preferred_element_type=f32) + cb2_ref[...])
    ph = jnp.maximum(jnp.dot(pos_ref[...], pw1_ref[...],
                             preferred_element_type=f32) + pb1_ref[...], 0.0)
    pos = _rows_normalized(jnp.dot(ph, pw2_ref[...],
                                   preferred_element_type=f32) + pb2_ref[...])
    emb = (jnp.dot(ce, mwce_ref[...], preferred_element_type=f32)
           + jnp.dot(col, mwcol_ref[...], preferred_element_type=f32)
           + jnp.dot(pos, mwpos_ref[...], preferred_element_type=f32)
           + mb_ref[...])
    r, d = emb.shape
    sq = jnp.sum(emb * emb, axis=1, keepdims=True)
    pad = ea_ref.shape[1] - d - 1
    ea_ref[:, 0:d] = -2.0 * emb
    ea_ref[:, d:d + 1] = sq
    ea_ref[:, d + 1:] = jnp.zeros((r, pad), f32)
    ebm_ref[:, 0:d] = emb
    ebm_ref[:, d:d + 1] = jnp.ones((r, 1), f32)
    ebm_ref[:, d + 1:] = jnp.zeros((r, pad), f32)
    wd = ew1a_ref[...] - ew1b_ref[...]
    u_ref[...] = ((jnp.dot(emb, wd, preferred_element_type=f32)
                   + eb1_ref[...]) * eg1_ref[...] + ebt1_ref[...])
    v_ref[...] = jnp.dot(emb, ew1b_ref[...],
                         preferred_element_type=f32) * eg1_ref[...]


def _make_knn_body(npad, k, chunk, off):
    def body(ea_ref, x_ref, brow_ref, bcol_ref, clo_ref, chi_ref, idx_ref):
        f32 = jnp.float32
        i32 = jnp.int32
        step = pl.program_id(0)
        r = x_ref.shape[0]
        x = x_ref[...]
        bi = bcol_ref[...]
        rowid = ((off + step) * r
                 + jax.lax.broadcasted_iota(i32, (r, 1), 0)).astype(f32)
        init_d = jnp.full((r, k), jnp.inf, f32)
        init_i = jnp.broadcast_to(
            jax.lax.broadcasted_iota(i32, (1, k), 1).astype(f32), (r, k))

        def chunk_body(c, carry):
            top_d, top_i = carry
            base = clo_ref[step] * 128 + c * chunk
            ea_c = ea_ref[pl.ds(base, chunk), :]
            s = jax.lax.dot_general(x, ea_c, (((1,), (1,)), ((), ())),
                                    preferred_element_type=f32)
            bj = brow_ref[:, pl.ds(base, chunk)]
            cid = (base.astype(f32)
                   + jax.lax.broadcasted_iota(i32, (1, chunk), 1).astype(f32))
            dv = jnp.where((bi == bj) & (cid != rowid), s, jnp.inf)
            vals = jnp.concatenate([dv, top_d], axis=1)
            idxm = jnp.concatenate(
                [jnp.broadcast_to(cid, (r, chunk)), top_i], axis=1)
            nd, ni = [], []
            fpad = float(npad)
            for _ in range(k):
                m = jnp.min(vals, axis=1, keepdims=True)
                sel = jnp.min(jnp.where(vals == m, idxm, fpad),
                              axis=1, keepdims=True)
                nd.append(m)
                ni.append(sel)
                hit = idxm == sel
                vals = jnp.where(hit, jnp.inf, vals)
                idxm = jnp.where(hit, fpad, idxm)
            return (jnp.concatenate(nd, axis=1), jnp.concatenate(ni, axis=1))

        _, top_i = jax.lax.fori_loop(0, chi_ref[step],
                                     chunk_body, (init_d, init_i))
        idx_ref[...] = top_i.astype(jnp.int32)
    return body


def _make_edge_body(nsteps, nseg, final):
    def common(u_ref, vj_ref, bcol_ref, ew2_ref, eb2_ref, acc_ref):
        f32 = jnp.float32
        step = pl.program_id(0)

        @pl.when(step == 0)
        def _init():
            acc_ref[...] = jnp.full(acc_ref.shape, -jnp.inf, f32)

        u = u_ref[...]
        w2 = ew2_ref[...]
        x = None
        for kk in range(vj_ref.shape[1]):
            hk = jnp.maximum(u + vj_ref[:, kk, :], 0.0)
            hk = jnp.dot(hk, w2, preferred_element_type=f32)
            x = hk if x is None else jnp.maximum(x, hk)
        x = x + eb2_ref[...]
        bi = bcol_ref[...]
        for b in range(nseg):
            mb = jnp.max(jnp.where(bi == b, x, -jnp.inf),
                         axis=0, keepdims=True)
            acc_ref[b:b + 1, :] = jnp.maximum(acc_ref[b:b + 1, :], mb)

    if not final:
        def body(u_ref, vj_ref, bcol_ref, ew2_ref, eb2_ref, out_ref, acc_ref):
            common(u_ref, vj_ref, bcol_ref, ew2_ref, eb2_ref, acc_ref)

            @pl.when(pl.program_id(0) == nsteps - 1)
            def _fin():
                out_ref[...] = acc_ref[...]
        return body

    def body(u_ref, vj_ref, bcol_ref, ew2_ref, eb2_ref, pa_ref,
             lw1_ref, lb1_ref, lw2_ref, lb2_ref, out_ref, acc_ref):
        f32 = jnp.float32
        common(u_ref, vj_ref, bcol_ref, ew2_ref, eb2_ref, acc_ref)

        @pl.when(pl.program_id(0) == nsteps - 1)
        def _fin():
            p = jnp.maximum(pa_ref[...], acc_ref[...])
            h1 = jnp.maximum(
                jnp.dot(p, lw1_ref[...], preferred_element_type=f32)
                + lb1_ref[...], 0.0)
            o = (jnp.dot(h1, lw2_ref[...], preferred_element_type=f32)
                 + lb2_ref[...])
            out_ref[...] = o / jnp.maximum(
                jnp.sqrt(jnp.sum(o * o, axis=1, keepdims=True)), 1e-12)
    return body


def _gather_rows(v, idx_flat):
    num_idx = idx_flat.shape[1]
    d = v.shape[1]
    width = 128
    mesh = plsc.VectorSubcoreMesh(core_axis_name="c", subcore_axis_name="s")

    @pl.kernel(out_type=jax.ShapeDtypeStruct((num_idx, d), v.dtype),
               mesh=mesh)
    def gk(v_hbm, i_hbm, o_hbm):
        def gather_block(i_vmem, o_vmem):
            pltpu.sync_copy(v_hbm.at[i_vmem.at[0]], o_vmem)

        pltpu.emit_pipeline(
            gather_block,
            grid=(num_idx // width,),
            in_specs=[pl.BlockSpec((1, width), index_map=lambda i: (0, i))],
            out_specs=[pl.BlockSpec((width, d), index_map=lambda i: (i, 0))],
            core_axis_name=("c", "s"),
            dimension_semantics=(pltpu.PARALLEL,),
        )(i_hbm, o_hbm)

    return gk(v, idx_flat)


def kernel(class_table, pos_W1, pos_b1, pos_W2, pos_b2, col_W1, col_b1,
           col_W2, col_b2, merge_W, merge_b, edge_W1, edge_b1, edge_g1,
           edge_bt1, edge_W2, edge_b2, lin_W1, lin_b1, lin_W2, lin_b2,
           colors, positions, class_indices, batch):
    f32 = jnp.float32
    n = colors.shape[0]
    d = class_table.shape[1]
    da = 2 * d
    npad = ((n + _R1 - 1) // _R1) * _R1
    pad = npad - n

    colors_p = jnp.pad(colors.astype(f32), ((0, pad), (0, d - colors.shape[1])))
    pos_p = jnp.pad(positions.astype(f32),
                    ((0, pad), (0, d - positions.shape[1])))
    cls_p = jnp.pad(class_indices.astype(jnp.int32), (0, pad)).reshape(npad, 1)
    batch_p = jnp.pad(batch.astype(jnp.int32), (0, pad), constant_values=_B)
    bcol = batch_p.reshape(npad, 1)
    brow = batch_p.reshape(1, npad)

    cw1 = jnp.pad(col_W1, ((0, d - col_W1.shape[0]), (0, 0)))
    pw1 = jnp.pad(pos_W1, ((0, d - pos_W1.shape[0]), (0, 0)))
    mwce, mwcol, mwpos = merge_W[:d], merge_W[d:2 * d], merge_W[2 * d:]
    ew1a, ew1b = edge_W1[:d], edge_W1[d:]
    row = lambda x: x.reshape(1, -1)

    def fixed(a):
        return pl.BlockSpec(a.shape, lambda i: tuple(0 for _ in a.shape))

    hdim = col_W1.shape[1]

    grid1 = npad // _R1
    weights1 = (class_table, cw1, row(col_b1), col_W2, row(col_b2),
                pw1, row(pos_b1), pos_W2, row(pos_b2),
                mwce, mwcol, mwpos, row(merge_b),
                ew1a, ew1b, row(edge_b1), row(edge_g1), row(edge_bt1))
    ea, ebm, u, v = pl.pallas_call(
        _embed_body,
        grid=(grid1,),
        in_specs=[fixed(w) for w in weights1] + [
            pl.BlockSpec((_R1, d), lambda i: (i, 0)),
            pl.BlockSpec((_R1, d), lambda i: (i, 0)),
            pl.BlockSpec((_R1, 1), lambda i: (i, 0)),
        ],
        out_specs=[
            pl.BlockSpec((_R1, da), lambda i: (i, 0)),
            pl.BlockSpec((_R1, da), lambda i: (i, 0)),
            pl.BlockSpec((_R1, d), lambda i: (i, 0)),
            pl.BlockSpec((_R1, d), lambda i: (i, 0)),
        ],
        out_shape=[
            jax.ShapeDtypeStruct((npad + _C2, da), f32),
            jax.ShapeDtypeStruct((npad, da), f32),
            jax.ShapeDtypeStruct((npad, d), f32),
            jax.ShapeDtypeStruct((npad, d), f32),
        ],
        compiler_params=pltpu.CompilerParams(
            dimension_semantics=("arbitrary",)),
    )(*weights1, colors_p, pos_p, cls_p)

    grid2 = npad // _R2
    b2 = batch_p.reshape(grid2, _R2)
    lo = jnp.searchsorted(batch_p, b2[:, 0], side="left")
    hi = jnp.searchsorted(batch_p, b2[:, -1], side="right")
    clo = (lo // 128).astype(jnp.int32)
    chi = ((hi - clo * 128 + _C2 - 1) // _C2).astype(jnp.int32)
    brow2 = jnp.pad(brow, ((0, 0), (0, _C2)), constant_values=_B + 1)

    nh2 = grid2 // 2
    half = npad // 2

    def knn_half(off):
        return pl.pallas_call(
            _make_knn_body(npad, _K, _C2, off * nh2),
            grid=(nh2,),
            in_specs=[
                pl.BlockSpec((npad + _C2, da), lambda i: (0, 0)),
                pl.BlockSpec((_R2, da), lambda i: (i + off * nh2, 0)),
                pl.BlockSpec((1, npad + _C2), lambda i: (0, 0)),
                pl.BlockSpec((_R2, 1), lambda i: (i + off * nh2, 0)),
                pl.BlockSpec(memory_space=pltpu.SMEM),
                pl.BlockSpec(memory_space=pltpu.SMEM),
            ],
            out_specs=pl.BlockSpec((_R2, _K), lambda i: (i, 0)),
            out_shape=jax.ShapeDtypeStruct((half, _K), jnp.int32),
            compiler_params=pltpu.CompilerParams(
                dimension_semantics=("arbitrary",)),
        )(ea, ebm, brow2, bcol,
          jax.lax.dynamic_slice_in_dim(clo, off * nh2, nh2),
          jax.lax.dynamic_slice_in_dim(chi, off * nh2, nh2))

    grid3 = half // _R3
    weights3 = (edge_W2, row(edge_b2))
    weights4 = (lin_W1, row(lin_b1), lin_W2, row(lin_b2))

    def edge_half(off, vj_h, pool_prev):
        final = pool_prev is not None
        extra = ((pool_prev,) + weights4) if final else ()
        return pl.pallas_call(
            _make_edge_body(grid3, _B, final),
            grid=(grid3,),
            in_specs=[
                pl.BlockSpec((_R3, d), lambda i: (i + off * grid3, 0)),
                pl.BlockSpec((_R3, _K, d), lambda i: (i, 0, 0)),
                pl.BlockSpec((_R3, 1), lambda i: (i + off * grid3, 0)),
            ] + [fixed(w) for w in weights3 + extra],
            out_specs=pl.BlockSpec((_B, d), lambda i: (0, 0)),
            out_shape=jax.ShapeDtypeStruct((_B, d), f32),
            scratch_shapes=[pltpu.VMEM((_B, d), f32)],
            compiler_params=pltpu.CompilerParams(
                dimension_semantics=("arbitrary",)),
        )(u, vj_h, bcol, *weights3, *extra)

    idx_a = knn_half(0)
    idx_b = knn_half(1)
    vj_a = _gather_rows(v, idx_a.reshape(1, half * _K)).reshape(half, _K, d)
    vj_b = _gather_rows(v, idx_b.reshape(1, half * _K)).reshape(half, _K, d)
    pool_a = edge_half(0, vj_a, None)
    out = edge_half(1, vj_b, pool_a)

    return out


# speedup vs baseline: 1.0226x; 1.0226x over previous
import jax
import jax.numpy as jnp
from jax.experimental import pallas as pl
from jax.experimental.pallas import tpu as pltpu
from jax.experimental.pallas import tpu_sc as plsc

_K = 8
_B = 16
_R1 = 1024
_R2 = 512
_C2 = 512
_R3 = 512


def _rows_normalized(x):
    n = jnp.sqrt(jnp.sum(x * x, axis=-1, keepdims=True))
    return x / jnp.maximum(n, 1e-12)


def _embed_body(ct_ref, cw1_ref, cb1_ref, cw2_ref, cb2_ref,
                pw1_ref, pb1_ref, pw2_ref, pb2_ref,
                mwce_ref, mwcol_ref, mwpos_ref, mb_ref,
                ew1a_ref, ew1b_ref, eb1_ref, eg1_ref, ebt1_ref,
                colors_ref, pos_ref, cls_ref,
                ea_ref, ebm_ref, u_ref, v_ref):
    f32 = jnp.float32
    cls = cls_ref[...]
    onehot = (cls == jax.lax.broadcasted_iota(
        jnp.int32, (1, ct_ref.shape[0]), 1)).astype(f32)
    ce = _rows_normalized(jnp.dot(onehot, ct_ref[...],
                                  preferred_element_type=f32))
    ch = jnp.maximum(jnp.dot(colors_ref[...], cw1_ref[...],
                             preferred_element_type=f32) + cb1_ref[...], 0.0)
    col = _rows_normalized(jnp.dot(ch, cw2_ref[...],
                                   preferred_element_type=f32) + cb2_ref[...])
    ph = jnp.maximum(jnp.dot(pos_ref[...], pw1_ref[...],
                             preferred_element_type=f32) + pb1_ref[...], 0.0)
    pos = _rows_normalized(jnp.dot(ph, pw2_ref[...],
                                   preferred_element_type=f32) + pb2_ref[...])
    emb = (jnp.dot(ce, mwce_ref[...], preferred_element_type=f32)
           + jnp.dot(col, mwcol_ref[...], preferred_element_type=f32)
           + jnp.dot(pos, mwpos_ref[...], preferred_element_type=f32)
           + mb_ref[...])
    r, d = emb.shape
    sq = jnp.sum(emb * emb, axis=1, keepdims=True)
    pad = ea_ref.shape[1] - d - 1
    ea_ref[:, 0:d] = -2.0 * emb
    ea_ref[:, d:d + 1] = sq
    ea_ref[:, d + 1:] = jnp.zeros((r, pad), f32)
    ebm_ref[:, 0:d] = emb
    ebm_ref[:, d:d + 1] = jnp.ones((r, 1), f32)
    ebm_ref[:, d + 1:] = jnp.zeros((r, pad), f32)
    wd = ew1a_ref[...] - ew1b_ref[...]
    u_ref[...] = ((jnp.dot(emb, wd, preferred_element_type=f32)
                   + eb1_ref[...]) * eg1_ref[...] + ebt1_ref[...])
    v_ref[...] = jnp.dot(emb, ew1b_ref[...],
                         preferred_element_type=f32) * eg1_ref[...]


def _make_knn_body(npad, k, chunk, off):
    def body(ea_ref, x_ref, brow_ref, bcol_ref, clo_ref, chi_ref, idx_ref):
        f32 = jnp.float32
        i32 = jnp.int32
        step = pl.program_id(0)
        r = x_ref.shape[0]
        x = x_ref[...]
        bi = bcol_ref[...]
        rowid = ((off + step) * r
                 + jax.lax.broadcasted_iota(i32, (r, 1), 0)).astype(f32)
        init_d = jnp.full((r, k), jnp.inf, f32)
        init_i = jnp.broadcast_to(
            jax.lax.broadcasted_iota(i32, (1, k), 1).astype(f32), (r, k))

        def chunk_body(c, carry):
            top_d, top_i = carry
            base = clo_ref[step] * 128 + c * chunk
            ea_c = ea_ref[pl.ds(base, chunk), :]
            s = jax.lax.dot_general(x, ea_c, (((1,), (1,)), ((), ())),
                                    preferred_element_type=f32)
            bj = brow_ref[:, pl.ds(base, chunk)]
            cid = (base.astype(f32)
                   + jax.lax.broadcasted_iota(i32, (1, chunk), 1).astype(f32))
            dv = jnp.where((bi == bj) & (cid != rowid), s, jnp.inf)
            vals = jnp.concatenate([dv, top_d], axis=1)
            idxm = jnp.concatenate(
                [jnp.broadcast_to(cid, (r, chunk)), top_i], axis=1)
            nd, ni = [], []
            fpad = float(npad)
            for _ in range(k):
                m = jnp.min(vals, axis=1, keepdims=True)
                sel = jnp.min(jnp.where(vals == m, idxm, fpad),
                              axis=1, keepdims=True)
                nd.append(m)
                ni.append(sel)
                hit = idxm == sel
                vals = jnp.where(hit, jnp.inf, vals)
                idxm = jnp.where(hit, fpad, idxm)
            return (jnp.concatenate(nd, axis=1), jnp.concatenate(ni, axis=1))

        _, top_i = jax.lax.fori_loop(0, chi_ref[step],
                                     chunk_body, (init_d, init_i))
        idx_ref[...] = top_i.astype(jnp.int32)
    return body


def _make_edge_body(nsteps, nseg):
    def common(u_ref, vj_ref, bcol_ref, ew2_ref, eb2_ref, acc_ref):
        f32 = jnp.float32
        step = pl.program_id(0)

        @pl.when(step == 0)
        def _init():
            acc_ref[...] = jnp.full(acc_ref.shape, -jnp.inf, f32)

        u = u_ref[...]
        w2 = ew2_ref[...]
        x = None
        for kk in range(vj_ref.shape[1]):
            hk = jnp.maximum(u + vj_ref[:, kk, :], 0.0)
            hk = jnp.dot(hk, w2, preferred_element_type=f32)
            x = hk if x is None else jnp.maximum(x, hk)
        x = x + eb2_ref[...]
        bi = bcol_ref[...]
        for b in range(nseg):
            mb = jnp.max(jnp.where(bi == b, x, -jnp.inf),
                         axis=0, keepdims=True)
            acc_ref[b:b + 1, :] = jnp.maximum(acc_ref[b:b + 1, :], mb)

    def body(u_ref, vj_ref, bcol_ref, ew2_ref, eb2_ref, out_ref, acc_ref):
        common(u_ref, vj_ref, bcol_ref, ew2_ref, eb2_ref, acc_ref)

        @pl.when(pl.program_id(0) == nsteps - 1)
        def _fin():
            out_ref[...] = acc_ref[...]
    return body


def _final_body(pa_ref, pb_ref, lw1_ref, lb1_ref, lw2_ref, lb2_ref, out_ref):
    f32 = jnp.float32
    p = jnp.maximum(pa_ref[...], pb_ref[...])
    h1 = jnp.maximum(jnp.dot(p, lw1_ref[...], preferred_element_type=f32)
                     + lb1_ref[...], 0.0)
    o = jnp.dot(h1, lw2_ref[...], preferred_element_type=f32) + lb2_ref[...]
    out_ref[...] = o / jnp.maximum(
        jnp.sqrt(jnp.sum(o * o, axis=1, keepdims=True)), 1e-12)


def _gather_rows(v, idx_flat):
    num_idx = idx_flat.shape[1]
    d = v.shape[1]
    width = 128
    mesh = plsc.VectorSubcoreMesh(core_axis_name="c", subcore_axis_name="s")

    @pl.kernel(out_type=jax.ShapeDtypeStruct((num_idx, d), v.dtype),
               mesh=mesh)
    def gk(v_hbm, i_hbm, o_hbm):
        def gather_block(i_vmem, o_vmem):
            pltpu.sync_copy(v_hbm.at[i_vmem.at[0]], o_vmem)

        pltpu.emit_pipeline(
            gather_block,
            grid=(num_idx // width,),
            in_specs=[pl.BlockSpec((1, width), index_map=lambda i: (0, i))],
            out_specs=[pl.BlockSpec((width, d), index_map=lambda i: (i, 0))],
            core_axis_name=("c", "s"),
            dimension_semantics=(pltpu.PARALLEL,),
        )(i_hbm, o_hbm)

    return gk(v, idx_flat)


def kernel(class_table, pos_W1, pos_b1, pos_W2, pos_b2, col_W1, col_b1,
           col_W2, col_b2, merge_W, merge_b, edge_W1, edge_b1, edge_g1,
           edge_bt1, edge_W2, edge_b2, lin_W1, lin_b1, lin_W2, lin_b2,
           colors, positions, class_indices, batch):
    f32 = jnp.float32
    n = colors.shape[0]
    d = class_table.shape[1]
    da = 2 * d
    npad = ((n + _R1 - 1) // _R1) * _R1
    pad = npad - n

    colors_p = jnp.pad(colors.astype(f32), ((0, pad), (0, d - colors.shape[1])))
    pos_p = jnp.pad(positions.astype(f32),
                    ((0, pad), (0, d - positions.shape[1])))
    cls_p = jnp.pad(class_indices.astype(jnp.int32), (0, pad)).reshape(npad, 1)
    batch_p = jnp.pad(batch.astype(jnp.int32), (0, pad), constant_values=_B)
    bcol = batch_p.reshape(npad, 1)
    brow = batch_p.reshape(1, npad)

    cw1 = jnp.pad(col_W1, ((0, d - col_W1.shape[0]), (0, 0)))
    pw1 = jnp.pad(pos_W1, ((0, d - pos_W1.shape[0]), (0, 0)))
    mwce, mwcol, mwpos = merge_W[:d], merge_W[d:2 * d], merge_W[2 * d:]
    ew1a, ew1b = edge_W1[:d], edge_W1[d:]
    row = lambda x: x.reshape(1, -1)

    def fixed(a):
        return pl.BlockSpec(a.shape, lambda i: tuple(0 for _ in a.shape))

    hdim = col_W1.shape[1]

    grid1 = npad // _R1
    weights1 = (class_table, cw1, row(col_b1), col_W2, row(col_b2),
                pw1, row(pos_b1), pos_W2, row(pos_b2),
                mwce, mwcol, mwpos, row(merge_b),
                ew1a, ew1b, row(edge_b1), row(edge_g1), row(edge_bt1))
    ea, ebm, u, v = pl.pallas_call(
        _embed_body,
        grid=(grid1,),
        in_specs=[fixed(w) for w in weights1] + [
            pl.BlockSpec((_R1, d), lambda i: (i, 0)),
            pl.BlockSpec((_R1, d), lambda i: (i, 0)),
            pl.BlockSpec((_R1, 1), lambda i: (i, 0)),
        ],
        out_specs=[
            pl.BlockSpec((_R1, da), lambda i: (i, 0)),
            pl.BlockSpec((_R1, da), lambda i: (i, 0)),
            pl.BlockSpec((_R1, d), lambda i: (i, 0)),
            pl.BlockSpec((_R1, d), lambda i: (i, 0)),
        ],
        out_shape=[
            jax.ShapeDtypeStruct((npad + _C2, da), f32),
            jax.ShapeDtypeStruct((npad, da), f32),
            jax.ShapeDtypeStruct((npad, d), f32),
            jax.ShapeDtypeStruct((npad, d), f32),
        ],
        compiler_params=pltpu.CompilerParams(
            dimension_semantics=("arbitrary",)),
    )(*weights1, colors_p, pos_p, cls_p)

    grid2 = npad // _R2
    b2 = batch_p.reshape(grid2, _R2)
    lo = jnp.searchsorted(batch_p, b2[:, 0], side="left")
    hi = jnp.searchsorted(batch_p, b2[:, -1], side="right")
    clo = (lo // 128).astype(jnp.int32)
    chi = ((hi - clo * 128 + _C2 - 1) // _C2).astype(jnp.int32)
    brow2 = jnp.pad(brow, ((0, 0), (0, _C2)), constant_values=_B + 1)

    nh2 = grid2 // 2
    half = npad // 2

    def knn_half(off):
        return pl.pallas_call(
            _make_knn_body(npad, _K, _C2, off * nh2),
            grid=(nh2,),
            in_specs=[
                pl.BlockSpec((npad + _C2, da), lambda i: (0, 0)),
                pl.BlockSpec((_R2, da), lambda i: (i + off * nh2, 0)),
                pl.BlockSpec((1, npad + _C2), lambda i: (0, 0)),
                pl.BlockSpec((_R2, 1), lambda i: (i + off * nh2, 0)),
                pl.BlockSpec(memory_space=pltpu.SMEM),
                pl.BlockSpec(memory_space=pltpu.SMEM),
            ],
            out_specs=pl.BlockSpec((_R2, _K), lambda i: (i, 0)),
            out_shape=jax.ShapeDtypeStruct((half, _K), jnp.int32),
            compiler_params=pltpu.CompilerParams(
                dimension_semantics=("arbitrary",)),
        )(ea, ebm, brow2, bcol,
          jax.lax.dynamic_slice_in_dim(clo, off * nh2, nh2),
          jax.lax.dynamic_slice_in_dim(chi, off * nh2, nh2))

    grid3 = half // _R3
    weights3 = (edge_W2, row(edge_b2))
    weights4 = (lin_W1, row(lin_b1), lin_W2, row(lin_b2))

    def edge_half(off, vj_h):
        return pl.pallas_call(
            _make_edge_body(grid3, _B),
            grid=(grid3,),
            in_specs=[
                pl.BlockSpec((_R3, d), lambda i: (i + off * grid3, 0)),
                pl.BlockSpec((_R3, _K, d), lambda i: (i, 0, 0)),
                pl.BlockSpec((_R3, 1), lambda i: (i + off * grid3, 0)),
            ] + [fixed(w) for w in weights3],
            out_specs=pl.BlockSpec((_B, d), lambda i: (0, 0)),
            out_shape=jax.ShapeDtypeStruct((_B, d), f32),
            scratch_shapes=[pltpu.VMEM((_B, d), f32)],
            compiler_params=pltpu.CompilerParams(
                dimension_semantics=("arbitrary",)),
        )(u, vj_h, bcol, *weights3)

    idx_a = knn_half(0)
    idx_b = knn_half(1)
    vj_a = _gather_rows(v, idx_a.reshape(1, half * _K)).reshape(half, _K, d)
    vj_b = _gather_rows(v, idx_b.reshape(1, half * _K)).reshape(half, _K, d)
    pool_a = edge_half(0, vj_a)
    pool_b = edge_half(1, vj_b)

    out = pl.pallas_call(
        _final_body,
        grid=(1,),
        in_specs=[fixed(pool_a), fixed(pool_b)] + [fixed(w) for w in weights4],
        out_specs=pl.BlockSpec((_B, d), lambda i: (0, 0)),
        out_shape=jax.ShapeDtypeStruct((_B, d), f32),
    )(pool_a, pool_b, *weights4)

    return out


# speedup vs baseline: 1.0234x; 1.0008x over previous
import jax
import jax.numpy as jnp
from jax.experimental import pallas as pl
from jax.experimental.pallas import tpu as pltpu
from jax.experimental.pallas import tpu_sc as plsc

_K = 8
_B = 16
_R1 = 1024
_R2 = 512
_C2 = 512
_R3 = 512


def _rows_normalized(x):
    n = jnp.sqrt(jnp.sum(x * x, axis=-1, keepdims=True))
    return x / jnp.maximum(n, 1e-12)


def _embed_body(ct_ref, cw1_ref, cb1_ref, cw2_ref, cb2_ref,
                pw1_ref, pb1_ref, pw2_ref, pb2_ref,
                mwce_ref, mwcol_ref, mwpos_ref, mb_ref,
                ew1a_ref, ew1b_ref, eb1_ref, eg1_ref, ebt1_ref,
                colors_ref, pos_ref, cls_ref,
                ea_ref, ebm_ref, u_ref, v_ref):
    f32 = jnp.float32
    cls = cls_ref[...]
    onehot = (cls == jax.lax.broadcasted_iota(
        jnp.int32, (1, ct_ref.shape[0]), 1)).astype(f32)
    ce = _rows_normalized(jnp.dot(onehot, ct_ref[...],
                                  preferred_element_type=f32))
    ch = jnp.maximum(jnp.dot(colors_ref[...], cw1_ref[...],
                             preferred_element_type=f32) + cb1_ref[...], 0.0)
    col = _rows_normalized(jnp.dot(ch, cw2_ref[...],
                                   preferred_element_type=f32) + cb2_ref[...])
    ph = jnp.maximum(jnp.dot(pos_ref[...], pw1_ref[...],
                             preferred_element_type=f32) + pb1_ref[...], 0.0)
    pos = _rows_normalized(jnp.dot(ph, pw2_ref[...],
                                   preferred_element_type=f32) + pb2_ref[...])
    emb = (jnp.dot(ce, mwce_ref[...], preferred_element_type=f32)
           + jnp.dot(col, mwcol_ref[...], preferred_element_type=f32)
           + jnp.dot(pos, mwpos_ref[...], preferred_element_type=f32)
           + mb_ref[...])
    r, d = emb.shape
    sq = jnp.sum(emb * emb, axis=1, keepdims=True)
    pad = ea_ref.shape[1] - d - 1
    ea_ref[:, 0:d] = -2.0 * emb
    ea_ref[:, d:d + 1] = sq
    ea_ref[:, d + 1:] = jnp.zeros((r, pad), f32)
    ebm_ref[:, 0:d] = emb
    ebm_ref[:, d:d + 1] = jnp.ones((r, 1), f32)
    ebm_ref[:, d + 1:] = jnp.zeros((r, pad), f32)
    wd = ew1a_ref[...] - ew1b_ref[...]
    u_ref[...] = ((jnp.dot(emb, wd, preferred_element_type=f32)
                   + eb1_ref[...]) * eg1_ref[...] + ebt1_ref[...])
    v_ref[...] = jnp.dot(emb, ew1b_ref[...],
                         preferred_element_type=f32) * eg1_ref[...]


def _make_knn_body(npad, k, chunk, off):
    def body(ea_ref, x_ref, brow_ref, bcol_ref, clo_ref, chi_ref, idx_ref):
        f32 = jnp.float32
        i32 = jnp.int32
        step = pl.program_id(0)
        r = x_ref.shape[0]
        x = x_ref[...]
        bi = bcol_ref[...]
        rowid = ((off + step) * r
                 + jax.lax.broadcasted_iota(i32, (r, 1), 0)).astype(f32)
        init_d = jnp.full((r, k), jnp.inf, f32)
        init_i = jnp.broadcast_to(
            jax.lax.broadcasted_iota(i32, (1, k), 1).astype(f32), (r, k))

        def chunk_body(c, carry):
            top_d, top_i = carry
            base = clo_ref[step] * 128 + c * chunk
            ea_c = ea_ref[pl.ds(base, chunk), :]
            s = jax.lax.dot_general(x, ea_c, (((1,), (1,)), ((), ())),
                                    preferred_element_type=f32)
            bj = brow_ref[:, pl.ds(base, chunk)]
            cid = (base.astype(f32)
                   + jax.lax.broadcasted_iota(i32, (1, chunk), 1).astype(f32))
            dv = jnp.where((bi == bj) & (cid != rowid), s, jnp.inf)
            vals = jnp.concatenate([dv, top_d], axis=1)
            idxm = jnp.concatenate(
                [jnp.broadcast_to(cid, (r, chunk)), top_i], axis=1)
            nd, ni = [], []
            fpad = float(npad)
            for t in range(k):
                m = jnp.min(vals, axis=1, keepdims=True)
                sel = jnp.min(jnp.where(vals == m, idxm, fpad),
                              axis=1, keepdims=True)
                nd.append(m)
                ni.append(sel)
                if t < k - 1:
                    hit = idxm == sel
                    vals = jnp.where(hit, jnp.inf, vals)
                    idxm = jnp.where(hit, fpad, idxm)
            return (jnp.concatenate(nd, axis=1), jnp.concatenate(ni, axis=1))

        _, top_i = jax.lax.fori_loop(0, chi_ref[step],
                                     chunk_body, (init_d, init_i))
        idx_ref[...] = top_i.astype(jnp.int32)
    return body


def _make_edge_body(nsteps, nseg):
    def common(u_ref, vj_ref, bcol_ref, ew2_ref, eb2_ref, acc_ref):
        f32 = jnp.float32
        step = pl.program_id(0)

        @pl.when(step == 0)
        def _init():
            acc_ref[...] = jnp.full(acc_ref.shape, -jnp.inf, f32)

        u = u_ref[...]
        w2 = ew2_ref[...]
        x = None
        for kk in range(vj_ref.shape[1]):
            hk = jnp.maximum(u + vj_ref[:, kk, :], 0.0)
            hk = jnp.dot(hk, w2, preferred_element_type=f32)
            x = hk if x is None else jnp.maximum(x, hk)
        x = x + eb2_ref[...]
        bi = bcol_ref[...]
        for b in range(nseg):
            mb = jnp.max(jnp.where(bi == b, x, -jnp.inf),
                         axis=0, keepdims=True)
            acc_ref[b:b + 1, :] = jnp.maximum(acc_ref[b:b + 1, :], mb)

    def body(u_ref, vj_ref, bcol_ref, ew2_ref, eb2_ref, out_ref, acc_ref):
        common(u_ref, vj_ref, bcol_ref, ew2_ref, eb2_ref, acc_ref)

        @pl.when(pl.program_id(0) == nsteps - 1)
        def _fin():
            out_ref[...] = acc_ref[...]
    return body


def _final_body(pa_ref, pb_ref, lw1_ref, lb1_ref, lw2_ref, lb2_ref, out_ref):
    f32 = jnp.float32
    p = jnp.maximum(pa_ref[...], pb_ref[...])
    h1 = jnp.maximum(jnp.dot(p, lw1_ref[...], preferred_element_type=f32)
                     + lb1_ref[...], 0.0)
    o = jnp.dot(h1, lw2_ref[...], preferred_element_type=f32) + lb2_ref[...]
    out_ref[...] = o / jnp.maximum(
        jnp.sqrt(jnp.sum(o * o, axis=1, keepdims=True)), 1e-12)


def _gather_rows(v, idx_flat):
    num_idx = idx_flat.shape[1]
    d = v.shape[1]
    width = 128
    mesh = plsc.VectorSubcoreMesh(core_axis_name="c", subcore_axis_name="s")

    @pl.kernel(out_type=jax.ShapeDtypeStruct((num_idx, d), v.dtype),
               mesh=mesh)
    def gk(v_hbm, i_hbm, o_hbm):
        def gather_block(i_vmem, o_vmem):
            pltpu.sync_copy(v_hbm.at[i_vmem.at[0]], o_vmem)

        pltpu.emit_pipeline(
            gather_block,
            grid=(num_idx // width,),
            in_specs=[pl.BlockSpec((1, width), index_map=lambda i: (0, i))],
            out_specs=[pl.BlockSpec((width, d), index_map=lambda i: (i, 0))],
            core_axis_name=("c", "s"),
            dimension_semantics=(pltpu.PARALLEL,),
        )(i_hbm, o_hbm)

    return gk(v, idx_flat)


def kernel(class_table, pos_W1, pos_b1, pos_W2, pos_b2, col_W1, col_b1,
           col_W2, col_b2, merge_W, merge_b, edge_W1, edge_b1, edge_g1,
           edge_bt1, edge_W2, edge_b2, lin_W1, lin_b1, lin_W2, lin_b2,
           colors, positions, class_indices, batch):
    f32 = jnp.float32
    n = colors.shape[0]
    d = class_table.shape[1]
    da = 2 * d
    npad = ((n + _R1 - 1) // _R1) * _R1
    pad = npad - n

    colors_p = jnp.pad(colors.astype(f32), ((0, pad), (0, d - colors.shape[1])))
    pos_p = jnp.pad(positions.astype(f32),
                    ((0, pad), (0, d - positions.shape[1])))
    cls_p = jnp.pad(class_indices.astype(jnp.int32), (0, pad)).reshape(npad, 1)
    batch_p = jnp.pad(batch.astype(jnp.int32), (0, pad), constant_values=_B)
    bcol = batch_p.reshape(npad, 1)
    brow = batch_p.reshape(1, npad)

    cw1 = jnp.pad(col_W1, ((0, d - col_W1.shape[0]), (0, 0)))
    pw1 = jnp.pad(pos_W1, ((0, d - pos_W1.shape[0]), (0, 0)))
    mwce, mwcol, mwpos = merge_W[:d], merge_W[d:2 * d], merge_W[2 * d:]
    ew1a, ew1b = edge_W1[:d], edge_W1[d:]
    row = lambda x: x.reshape(1, -1)

    def fixed(a):
        return pl.BlockSpec(a.shape, lambda i: tuple(0 for _ in a.shape))

    hdim = col_W1.shape[1]

    grid1 = npad // _R1
    weights1 = (class_table, cw1, row(col_b1), col_W2, row(col_b2),
                pw1, row(pos_b1), pos_W2, row(pos_b2),
                mwce, mwcol, mwpos, row(merge_b),
                ew1a, ew1b, row(edge_b1), row(edge_g1), row(edge_bt1))
    ea, ebm, u, v = pl.pallas_call(
        _embed_body,
        grid=(grid1,),
        in_specs=[fixed(w) for w in weights1] + [
            pl.BlockSpec((_R1, d), lambda i: (i, 0)),
            pl.BlockSpec((_R1, d), lambda i: (i, 0)),
            pl.BlockSpec((_R1, 1), lambda i: (i, 0)),
        ],
        out_specs=[
            pl.BlockSpec((_R1, da), lambda i: (i, 0)),
            pl.BlockSpec((_R1, da), lambda i: (i, 0)),
            pl.BlockSpec((_R1, d), lambda i: (i, 0)),
            pl.BlockSpec((_R1, d), lambda i: (i, 0)),
        ],
        out_shape=[
            jax.ShapeDtypeStruct((npad + _C2, da), f32),
            jax.ShapeDtypeStruct((npad, da), f32),
            jax.ShapeDtypeStruct((npad, d), f32),
            jax.ShapeDtypeStruct((npad, d), f32),
        ],
        compiler_params=pltpu.CompilerParams(
            dimension_semantics=("arbitrary",)),
    )(*weights1, colors_p, pos_p, cls_p)

    grid2 = npad // _R2
    b2 = batch_p.reshape(grid2, _R2)
    lo = jnp.searchsorted(batch_p, b2[:, 0], side="left")
    hi = jnp.searchsorted(batch_p, b2[:, -1], side="right")
    clo = (lo // 128).astype(jnp.int32)
    chi = ((hi - clo * 128 + _C2 - 1) // _C2).astype(jnp.int32)
    brow2 = jnp.pad(brow, ((0, 0), (0, _C2)), constant_values=_B + 1)

    nh2 = grid2 // 2
    half = npad // 2

    def knn_half(off):
        return pl.pallas_call(
            _make_knn_body(npad, _K, _C2, off * nh2),
            grid=(nh2,),
            in_specs=[
                pl.BlockSpec((npad + _C2, da), lambda i: (0, 0)),
                pl.BlockSpec((_R2, da), lambda i: (i + off * nh2, 0)),
                pl.BlockSpec((1, npad + _C2), lambda i: (0, 0)),
                pl.BlockSpec((_R2, 1), lambda i: (i + off * nh2, 0)),
                pl.BlockSpec(memory_space=pltpu.SMEM),
                pl.BlockSpec(memory_space=pltpu.SMEM),
            ],
            out_specs=pl.BlockSpec((_R2, _K), lambda i: (i, 0)),
            out_shape=jax.ShapeDtypeStruct((half, _K), jnp.int32),
            compiler_params=pltpu.CompilerParams(
                dimension_semantics=("arbitrary",)),
        )(ea, ebm, brow2, bcol,
          jax.lax.dynamic_slice_in_dim(clo, off * nh2, nh2),
          jax.lax.dynamic_slice_in_dim(chi, off * nh2, nh2))

    grid3 = half // _R3
    weights3 = (edge_W2, row(edge_b2))
    weights4 = (lin_W1, row(lin_b1), lin_W2, row(lin_b2))

    def edge_half(off, vj_h):
        return pl.pallas_call(
            _make_edge_body(grid3, _B),
            grid=(grid3,),
            in_specs=[
                pl.BlockSpec((_R3, d), lambda i: (i + off * grid3, 0)),
                pl.BlockSpec((_R3, _K, d), lambda i: (i, 0, 0)),
                pl.BlockSpec((_R3, 1), lambda i: (i + off * grid3, 0)),
            ] + [fixed(w) for w in weights3],
            out_specs=pl.BlockSpec((_B, d), lambda i: (0, 0)),
            out_shape=jax.ShapeDtypeStruct((_B, d), f32),
            scratch_shapes=[pltpu.VMEM((_B, d), f32)],
            compiler_params=pltpu.CompilerParams(
                dimension_semantics=("arbitrary",)),
        )(u, vj_h, bcol, *weights3)

    idx_a = knn_half(0)
    idx_b = knn_half(1)
    vj_a = _gather_rows(v, idx_a.reshape(1, half * _K)).reshape(half, _K, d)
    vj_b = _gather_rows(v, idx_b.reshape(1, half * _K)).reshape(half, _K, d)
    pool_a = edge_half(0, vj_a)
    pool_b = edge_half(1, vj_b)

    out = pl.pallas_call(
        _final_body,
        grid=(1,),
        in_specs=[fixed(pool_a), fixed(pool_b)] + [fixed(w) for w in weights4],
        out_specs=pl.BlockSpec((_B, d), lambda i: (0, 0)),
        out_shape=jax.ShapeDtypeStruct((_B, d), f32),
    )(pool_a, pool_b, *weights4)

    return out
